# Initial kernel scaffold; baseline (speedup 1.0000x reference)
import functools

import jax
import jax.numpy as jnp
from jax import lax
from jax.experimental import pallas as pl
from jax.experimental.pallas import tpu as pltpu
from jax.experimental.pallas import tpu_sc as plsc

N = 10000
E = 320000
D = 128
DE = 16

NC = 2
NS = 16
NW = NC * NS
EW = E // NW
NP = 10240
T = NP // NS
EC = EW // 16
CH = 128
NCH = EW // CH
TAIL = EW - NCH * CH

_mesh = plsc.VectorSubcoreMesh(
    core_axis_name="c", subcore_axis_name="s", num_cores=NC, num_subcores=NS)

f32 = jnp.float32


def _zst_body(x_ref, wfcT_ref, bfc_ref, wst_ref, z_ref, st_ref):
    z = jnp.dot(x_ref[...], wfcT_ref[...], preferred_element_type=f32)
    z = z + bfc_ref[...]
    z_ref[...] = z
    st_ref[...] = jnp.dot(z, wst_ref[...], preferred_element_type=f32)


def _tc_zst(x, wfcT, bfc2, wst):
    blk = 400
    return pl.pallas_call(
        _zst_body,
        grid=(N // blk,),
        in_specs=[
            pl.BlockSpec((blk, D), lambda i: (i, 0)),
            pl.BlockSpec((D, D), lambda i: (0, 0)),
            pl.BlockSpec((1, D), lambda i: (0, 0)),
            pl.BlockSpec((D, D), lambda i: (0, 0)),
        ],
        out_specs=[
            pl.BlockSpec((blk, D), lambda i: (i, 0)),
            pl.BlockSpec((blk, D), lambda i: (i, 0)),
        ],
        out_shape=[
            jax.ShapeDtypeStruct((N, D), f32),
            jax.ShapeDtypeStruct((N, D), f32),
        ],
    )(x, wfcT, bfc2, wst)


def _u_body(ea_ref, v_ref, c_ref, u_ref):
    u_ref[...] = jnp.dot(ea_ref[...], v_ref[...],
                         preferred_element_type=f32) + c_ref[...]


def _tc_u(ea2, V128, c2):
    rows = E // 128
    blk = 100
    return pl.pallas_call(
        _u_body,
        grid=(rows // blk,),
        in_specs=[
            pl.BlockSpec((blk, 2048), lambda i: (i, 0)),
            pl.BlockSpec((2048, 128), lambda i: (0, 0)),
            pl.BlockSpec((1, 1), lambda i: (0, 0)),
        ],
        out_specs=pl.BlockSpec((blk, 128), lambda i: (i, 0)),
        out_shape=jax.ShapeDtypeStruct((rows, 128), f32),
    )(ea2, V128, c2)


def _fin_body(h_ref, s_ref, d_ref, w_ref, b_ref, o_ref):
    hp = h_ref[0] + h_ref[1]
    sp = s_ref[0] + s_ref[1]
    dt = d_ref[0] + d_ref[1]
    asum = jnp.where(dt > 0, 1.0, 0.0)
    o_ref[...] = hp + jnp.dot(sp, w_ref[...],
                              preferred_element_type=f32) + asum * b_ref[...]


def _tc_final(hpart, spart, den3, wedgeT, bedge2):
    blk = 640
    return pl.pallas_call(
        _fin_body,
        grid=(NP // blk,),
        in_specs=[
            pl.BlockSpec((2, blk, D), lambda i: (0, i, 0)),
            pl.BlockSpec((2, blk, DE), lambda i: (0, i, 0)),
            pl.BlockSpec((2, blk, 1), lambda i: (0, i, 0)),
            pl.BlockSpec((DE, D), lambda i: (0, 0)),
            pl.BlockSpec((1, D), lambda i: (0, 0)),
        ],
        out_specs=pl.BlockSpec((blk, D), lambda i: (i, 0)),
        out_shape=jax.ShapeDtypeStruct((NP, D), f32),
    )(hpart, spart, den3, wedgeT, bedge2)


@functools.partial(
    pl.kernel,
    out_type=[
        jax.ShapeDtypeStruct((E,), f32),
        jax.ShapeDtypeStruct((NC * NP,), f32),
    ],
    mesh=_mesh,
    scratch_types=[
        pltpu.VMEM((EW,), jnp.int32),
        pltpu.VMEM((EW,), jnp.int32),
        pltpu.VMEM((EW,), f32),
        pltpu.VMEM((NP,), f32),
        pltpu.VMEM((NP,), f32),
        pltpu.VMEM((EW,), f32),
        pltpu.VMEM((NP,), f32),
        pltpu.VMEM((T,), f32),
        pltpu.VMEM((T,), f32),
        pltpu.VMEM_SHARED((NS, NP), f32),
    ],
)
def _sc_pass1(src_hbm, dst_hbm, u_hbm, s_hbm, t_hbm, p_hbm, den_hbm,
              src_v, dst_v, u_v, s_v, t_v, p_v, den_v, acc_v, tmp_v, den_sh):
    cid = lax.axis_index("c")
    sid = lax.axis_index("s")
    wid = sid * NC + cid
    base = wid * EW

    pltpu.sync_copy(src_hbm.at[pl.ds(base, EW)], src_v)
    pltpu.sync_copy(dst_hbm.at[pl.ds(base, EW)], dst_v)
    pltpu.sync_copy(u_hbm.at[pl.ds(base, EW)], u_v)
    pltpu.sync_copy(s_hbm, s_v)
    pltpu.sync_copy(t_hbm, t_v)

    zeros16 = jnp.zeros((16,), f32)

    def zero_body(i, c):
        den_v[pl.ds(i * 16, 16)] = zeros16
        return c
    lax.fori_loop(0, NP // 16, zero_body, 0)

    def edge_body(i, c):
        sl = pl.ds(i * 16, 16)
        src16 = src_v[sl]
        dst16 = dst_v[sl]
        sv = plsc.load_gather(s_v, [src16])
        tv = plsc.load_gather(t_v, [dst16])
        a = sv + tv + u_v[sl]
        e = jnp.where(a >= 0, a, a * 0.2)
        p = jnp.exp(e)
        p_v[sl] = p
        plsc.addupdate_scatter(den_v, [dst16], p)
        return c
    lax.fori_loop(0, EC, edge_body, 0)

    pltpu.sync_copy(p_v, p_hbm.at[pl.ds(base, EW)])

    pltpu.sync_copy(den_v, den_sh.at[sid])
    plsc.subcore_barrier()
    off = sid * T
    pltpu.sync_copy(den_sh.at[0, pl.ds(off, T)], acc_v)
    for k in range(1, NS):
        pltpu.sync_copy(den_sh.at[k, pl.ds(off, T)], tmp_v)

        def add_body(i, c):
            sl = pl.ds(i * 16, 16)
            acc_v[sl] = acc_v[sl] + tmp_v[sl]
            return c
        lax.fori_loop(0, T // 16, add_body, 0)
    pltpu.sync_copy(acc_v, den_hbm.at[pl.ds(cid * NP + off, T)])


@functools.partial(
    pl.kernel,
    out_type=[
        jax.ShapeDtypeStruct((E,), f32),
        jax.ShapeDtypeStruct((NC * NP, D), f32),
        jax.ShapeDtypeStruct((NC * NP, DE), f32),
    ],
    mesh=_mesh,
    scratch_types=[
        pltpu.VMEM((EW,), jnp.int32),
        pltpu.VMEM((EW,), jnp.int32),
        pltpu.VMEM((EW,), f32),
        pltpu.VMEM((NP,), f32),
        pltpu.VMEM((NP,), f32),
        pltpu.VMEM((CH, D), f32),
        pltpu.VMEM((CH, DE), f32),
        pltpu.VMEM((CH,), jnp.int32),
        pltpu.VMEM((CH,), jnp.int32),
        pltpu.VMEM((TAIL,), jnp.int32),
        pltpu.VMEM((TAIL,), jnp.int32),
        pltpu.VMEM_SHARED((NP, D), f32),
        pltpu.VMEM_SHARED((NP, DE), f32),
        pltpu.SemaphoreType.DMA,
    ],
)
def _sc_pass2(src_hbm, dst_hbm, p_hbm, den_hbm, z_hbm, ea_hbm,
              alpha_hbm, h_hbm, s16_hbm,
              src_v, dst_v, al_v, rden_v, d2_v, rows_v, ea_v,
              sidx_v, didx_v, sidx_t, didx_t, h_sh, s_sh, sem):
    cid = lax.axis_index("c")
    sid = lax.axis_index("s")
    wid = sid * NC + cid
    base = wid * EW

    pltpu.sync_copy(src_hbm.at[pl.ds(base, EW)], src_v)
    pltpu.sync_copy(dst_hbm.at[pl.ds(base, EW)], dst_v)
    pltpu.sync_copy(p_hbm.at[pl.ds(base, EW)], al_v)
    pltpu.sync_copy(den_hbm.at[pl.ds(0, NP)], rden_v)
    pltpu.sync_copy(den_hbm.at[pl.ds(NP, NP)], d2_v)

    def rden_body(i, c):
        sl = pl.ds(i * 16, 16)
        rden_v[sl] = 1.0 / (rden_v[sl] + d2_v[sl])
        return c
    lax.fori_loop(0, NP // 16, rden_body, 0)

    def alpha_body(i, c):
        sl = pl.ds(i * 16, 16)
        rd = plsc.load_gather(rden_v, [dst_v[sl]])
        al_v[sl] = al_v[sl] * rd
        return c
    lax.fori_loop(0, EC, alpha_body, 0)

    pltpu.sync_copy(al_v, alpha_hbm.at[pl.ds(base, EW)])

    zeros16 = jnp.zeros((16,), f32)

    def zero_body(j, c):
        for r in range(D // 16):
            rows_v[j, pl.ds(r * 16, 16)] = zeros16
        ea_v[j, pl.ds(0, 16)] = zeros16
        return c
    lax.fori_loop(0, CH, zero_body, 0)

    for k in range(T // CH):
        off = sid * T + k * CH
        pltpu.sync_copy(rows_v, h_sh.at[pl.ds(off, CH)])
        pltpu.sync_copy(ea_v, s_sh.at[pl.ds(off, CH)])
    plsc.subcore_barrier()

    def scale_rows(nrows, cb):
        def sbody(j, c2):
            al = al_v[cb + j]
            av = jnp.full((16,), al, f32)
            for r in range(D // 16):
                sl = pl.ds(r * 16, 16)
                rows_v[j, sl] = rows_v[j, sl] * av
            ea_v[j, pl.ds(0, 16)] = ea_v[j, pl.ds(0, 16)] * av
            return c2
        lax.fori_loop(0, nrows, sbody, 0)

    def chunk_body(c, carry):
        cb = c * CH
        pltpu.sync_copy(src_v.at[pl.ds(cb, CH)], sidx_v)
        pltpu.sync_copy(dst_v.at[pl.ds(cb, CH)], didx_v)
        pltpu.async_copy(z_hbm.at[sidx_v], rows_v, sem).wait()
        pltpu.sync_copy(ea_hbm.at[pl.ds(base + cb, CH)], ea_v)
        scale_rows(CH, cb)
        pltpu.sync_copy(rows_v, h_sh.at[didx_v], add=True)
        pltpu.sync_copy(ea_v, s_sh.at[didx_v], add=True)
        return carry
    lax.fori_loop(0, NCH, chunk_body, 0)

    tb = NCH * CH
    pltpu.sync_copy(src_v.at[pl.ds(tb, TAIL)], sidx_t)
    pltpu.sync_copy(dst_v.at[pl.ds(tb, TAIL)], didx_t)
    pltpu.async_copy(z_hbm.at[sidx_t], rows_v.at[pl.ds(0, TAIL)], sem).wait()
    pltpu.sync_copy(ea_hbm.at[pl.ds(base + tb, TAIL)], ea_v.at[pl.ds(0, TAIL)])
    scale_rows(TAIL, tb)
    pltpu.sync_copy(rows_v.at[pl.ds(0, TAIL)], h_sh.at[didx_t], add=True)
    pltpu.sync_copy(ea_v.at[pl.ds(0, TAIL)], s_sh.at[didx_t], add=True)

    plsc.subcore_barrier()
    off = sid * T
    pltpu.sync_copy(h_sh.at[pl.ds(off, T)], h_hbm.at[pl.ds(cid * NP + off, T)])
    pltpu.sync_copy(s_sh.at[pl.ds(off, T)], s16_hbm.at[pl.ds(cid * NP + off, T)])


def kernel(x, edge_index, edge_attr, W_fc, b_fc, W_attn, b_attn,
           W_edge, b_edge, W_eatt, b_eatt):
    w1 = W_attn[0, :D]
    w2 = W_attn[0, D:2 * D]
    w3 = W_attn[0, 2 * D:]
    wfcT = W_fc.T
    bfc2 = b_fc.reshape(1, D)
    wst = jnp.zeros((D, D), f32).at[:, 0].set(w1).at[:, 1].set(w2)
    v_att = W_eatt.T @ w3
    c2 = (jnp.dot(b_eatt, w3) + b_attn[0]).reshape(1, 1)
    V128 = jnp.kron(jnp.eye(128, dtype=f32), v_att[:, None])
    wedgeT = W_edge.T
    bedge2 = b_edge.reshape(1, D)

    z, st = _tc_zst(x, wfcT, bfc2, wst)
    u2 = _tc_u(edge_attr.reshape(E // 128, 128 * DE), V128, c2)
    u = u2.reshape(E)
    s = jnp.pad(st[:, 0], (0, NP - N))
    t = jnp.pad(st[:, 1], (0, NP - N))

    src = edge_index[0]
    dst = edge_index[1]
    p, den = _sc_pass1(src, dst, u, s, t)
    alpha, hpart, spart = _sc_pass2(src, dst, p, den, z, edge_attr)

    h = _tc_final(hpart.reshape(NC, NP, D), spart.reshape(NC, NP, DE),
                  den.reshape(NC, NP, 1), wedgeT, bedge2)
    return h[:N], alpha.reshape(E, 1)


# speedup vs baseline: 5.9819x; 5.9819x over previous
import functools

import jax
import jax.numpy as jnp
from jax import lax
from jax.experimental import pallas as pl
from jax.experimental.pallas import tpu as pltpu
from jax.experimental.pallas import tpu_sc as plsc

N = 10000
E = 320000
D = 128
DE = 16

NC = 2
NS = 16
NW = NC * NS
EW = E // NW
NP = 10240
T = NP // NS
EC = EW // 16
CH = 64
NCH = EW // CH
TAIL = EW - NCH * CH

_mesh = plsc.VectorSubcoreMesh(
    core_axis_name="c", subcore_axis_name="s", num_cores=NC, num_subcores=NS)

f32 = jnp.float32


def _zst_body(x_ref, wfcT_ref, bfc_ref, wst_ref, z_ref, st_ref):
    z = jnp.dot(x_ref[...], wfcT_ref[...], preferred_element_type=f32)
    z = z + bfc_ref[...]
    z_ref[...] = z
    st_ref[...] = jnp.dot(z, wst_ref[...], preferred_element_type=f32)


def _tc_zst(x, wfcT, bfc2, wst):
    blk = 400
    return pl.pallas_call(
        _zst_body,
        grid=(N // blk,),
        in_specs=[
            pl.BlockSpec((blk, D), lambda i: (i, 0)),
            pl.BlockSpec((D, D), lambda i: (0, 0)),
            pl.BlockSpec((1, D), lambda i: (0, 0)),
            pl.BlockSpec((D, D), lambda i: (0, 0)),
        ],
        out_specs=[
            pl.BlockSpec((blk, D), lambda i: (i, 0)),
            pl.BlockSpec((blk, D), lambda i: (i, 0)),
        ],
        out_shape=[
            jax.ShapeDtypeStruct((N, D), f32),
            jax.ShapeDtypeStruct((N, D), f32),
        ],
    )(x, wfcT, bfc2, wst)


def _u_body(ea_ref, v_ref, c_ref, u_ref):
    u_ref[...] = jnp.dot(ea_ref[...], v_ref[...],
                         preferred_element_type=f32) + c_ref[...]


def _tc_u(ea2, V128, c2):
    rows = E // 128
    blk = rows
    return pl.pallas_call(
        _u_body,
        grid=(rows // blk,),
        in_specs=[
            pl.BlockSpec((blk, 2048), lambda i: (i, 0)),
            pl.BlockSpec((2048, 128), lambda i: (0, 0)),
            pl.BlockSpec((1, 1), lambda i: (0, 0)),
        ],
        out_specs=pl.BlockSpec((blk, 128), lambda i: (i, 0)),
        out_shape=jax.ShapeDtypeStruct((rows, 128), f32),
    )(ea2, V128, c2)


def _ez_body(ea_ref, w_ref, b_ref, o_ref):
    o_ref[...] = jnp.dot(ea_ref[...], w_ref[...],
                         preferred_element_type=f32) + b_ref[...]


def _tc_ez(edge_attr, wedgeT, bedge2):
    blk = 3200
    return pl.pallas_call(
        _ez_body,
        grid=(E // blk,),
        in_specs=[
            pl.BlockSpec((blk, DE), lambda i: (i, 0)),
            pl.BlockSpec((DE, D), lambda i: (0, 0)),
            pl.BlockSpec((1, D), lambda i: (0, 0)),
        ],
        out_specs=pl.BlockSpec((blk, D), lambda i: (i, 0)),
        out_shape=jax.ShapeDtypeStruct((E, D), f32),
    )(edge_attr, wedgeT, bedge2)


def _fin_body(h_ref, o_ref):
    o_ref[...] = h_ref[0] + h_ref[1]


def _tc_final(hpart):
    blk = 640
    return pl.pallas_call(
        _fin_body,
        grid=(NP // blk,),
        in_specs=[pl.BlockSpec((2, blk, D), lambda i: (0, i, 0))],
        out_specs=pl.BlockSpec((blk, D), lambda i: (i, 0)),
        out_shape=jax.ShapeDtypeStruct((NP, D), f32),
    )(hpart)


@functools.partial(
    pl.kernel,
    out_type=[
        jax.ShapeDtypeStruct((E,), f32),
        jax.ShapeDtypeStruct((NC * NP,), f32),
    ],
    mesh=_mesh,
    compiler_params=pltpu.CompilerParams(needs_layout_passes=False),
    scratch_types=[
        pltpu.VMEM((EW,), jnp.int32),
        pltpu.VMEM((EW,), jnp.int32),
        pltpu.VMEM((EW,), f32),
        pltpu.VMEM((NP,), f32),
        pltpu.VMEM((NP,), f32),
        pltpu.VMEM((EW,), f32),
        pltpu.VMEM((NP,), f32),
        pltpu.VMEM((T,), f32),
        pltpu.VMEM((T,), f32),
        pltpu.VMEM_SHARED((NS, NP), f32),
    ],
)
def _sc_pass1(src_hbm, dst_hbm, u_hbm, s_hbm, t_hbm, p_hbm, den_hbm,
              src_v, dst_v, u_v, s_v, t_v, p_v, den_v, acc_v, tmp_v, den_sh):
    cid = lax.axis_index("c")
    sid = lax.axis_index("s")
    wid = sid * NC + cid
    base = wid * EW

    pltpu.sync_copy(src_hbm.at[pl.ds(base, EW)], src_v)
    pltpu.sync_copy(dst_hbm.at[pl.ds(base, EW)], dst_v)
    pltpu.sync_copy(u_hbm.at[pl.ds(base, EW)], u_v)
    pltpu.sync_copy(s_hbm, s_v)
    pltpu.sync_copy(t_hbm, t_v)

    zeros16 = jnp.zeros((16,), f32)

    def zero_body(i, c):
        den_v[pl.ds(i * 16, 16)] = zeros16
        return c
    lax.fori_loop(0, NP // 16, zero_body, 0)

    def edge_body(i, c):
        sl = pl.ds(i * 16, 16)
        src16 = src_v[sl]
        dst16 = dst_v[sl]
        sv = plsc.load_gather(s_v, [src16])
        tv = plsc.load_gather(t_v, [dst16])
        a = sv + tv + u_v[sl]
        e = jnp.where(a >= 0, a, a * 0.2)
        p = jnp.exp(e)
        p_v[sl] = p
        plsc.addupdate_scatter(den_v, [dst16], p)
        return c
    lax.fori_loop(0, EC, edge_body, 0)

    pltpu.sync_copy(p_v, p_hbm.at[pl.ds(base, EW)])

    pltpu.sync_copy(den_v, den_sh.at[sid])
    plsc.subcore_barrier()
    off = sid * T
    pltpu.sync_copy(den_sh.at[0, pl.ds(off, T)], acc_v)
    for k in range(1, NS):
        pltpu.sync_copy(den_sh.at[k, pl.ds(off, T)], tmp_v)

        def add_body(i, c):
            sl = pl.ds(i * 16, 16)
            acc_v[sl] = acc_v[sl] + tmp_v[sl]
            return c
        lax.fori_loop(0, T // 16, add_body, 0)
    pltpu.sync_copy(acc_v, den_hbm.at[pl.ds(cid * NP + off, T)])


def _rden_body(d_ref, o_ref):
    o_ref[...] = 1.0 / (d_ref[0] + d_ref[1])


def _tc_rden(den3):
    return pl.pallas_call(
        _rden_body,
        grid=(1,),
        in_specs=[pl.BlockSpec((2, NP // 128, 128), lambda i: (0, 0, 0))],
        out_specs=pl.BlockSpec((NP // 128, 128), lambda i: (0, 0)),
        out_shape=jax.ShapeDtypeStruct((NP // 128, 128), f32),
    )(den3)


@functools.partial(
    pl.kernel,
    out_type=[
        jax.ShapeDtypeStruct((E,), f32),
        jax.ShapeDtypeStruct((NC * NP, D), f32),
    ],
    mesh=_mesh,
    compiler_params=pltpu.CompilerParams(needs_layout_passes=False),
    scratch_types=[
        pltpu.VMEM((NP,), f32),
        pltpu.VMEM((CH,), f32),
        pltpu.VMEM((CH, D), f32),
        pltpu.VMEM((CH, D), f32),
        pltpu.VMEM((CH,), jnp.int32),
        pltpu.VMEM((CH,), jnp.int32),
        pltpu.VMEM((TAIL,), jnp.int32),
        pltpu.VMEM((TAIL,), jnp.int32),
        pltpu.VMEM_SHARED((NP, D), f32),
        pltpu.SemaphoreType.DMA,
    ],
)
def _sc_pass2(src_hbm, dst_hbm, p_hbm, rden_hbm, z_hbm, ez_hbm, zh_hbm,
              alpha_hbm, h_hbm,
              rden_v, al_c, rows_v, ezr_v,
              sidx_v, didx_v, sidx_t, didx_t, h_sh, sem):
    cid = lax.axis_index("c")
    sid = lax.axis_index("s")
    wid = sid * NC + cid
    base = wid * EW

    pltpu.sync_copy(rden_hbm, rden_v)

    @pl.when(sid == 0)
    def _zero():
        pltpu.sync_copy(zh_hbm, h_sh)
    plsc.subcore_barrier()

    def do_chunk(nrows, cb, sbuf, dbuf):
        gsl = pl.ds(base + cb, nrows)
        pltpu.sync_copy(src_hbm.at[gsl], sbuf)
        pltpu.sync_copy(dst_hbm.at[gsl], dbuf)
        rows = rows_v if nrows == CH else rows_v.at[pl.ds(0, nrows)]
        ezr = ezr_v if nrows == CH else ezr_v.at[pl.ds(0, nrows)]
        pltpu.async_copy(z_hbm.at[sbuf], rows, sem).wait()
        pltpu.sync_copy(ez_hbm.at[gsl], ezr)
        pltpu.sync_copy(p_hbm.at[gsl], al_c.at[pl.ds(0, nrows)])

        def albody(r, c2):
            sl = pl.ds(r * 16, 16)
            rd = plsc.load_gather(rden_v, [dbuf[sl]])
            al_c[sl] = al_c[sl] * rd
            return c2
        lax.fori_loop(0, nrows // 16, albody, 0)
        pltpu.sync_copy(al_c.at[pl.ds(0, nrows)], alpha_hbm.at[gsl])

        def sbody(j, c2):
            av = plsc.load_gather(al_c, [jnp.full((16,), j, jnp.int32)])
            for r in range(D // 16):
                sl = pl.ds(r * 16, 16)
                rows_v[j, sl] = (rows_v[j, sl] + ezr_v[j, sl]) * av
            return c2
        lax.fori_loop(0, nrows, sbody, 0)

        pltpu.sync_copy(rows, h_sh.at[dbuf], add=True)

    def chunk_body(c, carry):
        do_chunk(CH, c * CH, sidx_v, didx_v)
        return carry
    lax.fori_loop(0, NCH, chunk_body, 0)

    do_chunk(TAIL, NCH * CH, sidx_t, didx_t)

    plsc.subcore_barrier()

    @pl.when(sid == 0)
    def _dump():
        pltpu.sync_copy(h_sh, h_hbm.at[pl.ds(cid * NP, NP)])


def kernel(x, edge_index, edge_attr, W_fc, b_fc, W_attn, b_attn,
           W_edge, b_edge, W_eatt, b_eatt):
    w1 = W_attn[0, :D]
    w2 = W_attn[0, D:2 * D]
    w3 = W_attn[0, 2 * D:]
    wfcT = W_fc.T
    bfc2 = b_fc.reshape(1, D)
    wst = jnp.zeros((D, D), f32).at[:, 0].set(w1).at[:, 1].set(w2)
    v_att = W_eatt.T @ w3
    c2 = (jnp.dot(b_eatt, w3) + b_attn[0]).reshape(1, 1)
    V128 = jnp.kron(jnp.eye(128, dtype=f32), v_att[:, None])
    wedgeT = W_edge.T
    bedge2 = b_edge.reshape(1, D)

    z, st = _tc_zst(x, wfcT, bfc2, wst)
    u2 = _tc_u(edge_attr.reshape(E // 128, 128 * DE), V128, c2)
    u = u2.reshape(E)
    ez = _tc_ez(edge_attr, wedgeT, bedge2)
    s = jnp.pad(st[:, 0], (0, NP - N))
    t = jnp.pad(st[:, 1], (0, NP - N))
    zh0 = jnp.zeros((NP, D), f32)

    src = edge_index[0]
    dst = edge_index[1]
    p, den = _sc_pass1(src, dst, u, s, t)
    rden = _tc_rden(den.reshape(NC, NP // 128, 128)).reshape(NP)
    alpha, hpart = _sc_pass2(src, dst, p, rden, z, ez, zh0)

    h = _tc_final(hpart.reshape(NC, NP, D))
    return h[:N], alpha.reshape(E, 1)
